# baseline (speedup 1.0000x reference)
import functools

import jax
import jax.numpy as jnp
from jax import lax
from jax.experimental import pallas as pl
from jax.experimental.pallas import tpu as pltpu
from jax.experimental.pallas import tpu_sc as plsc

_B, _S, _N, _F = 16, 128, 2048, 8
_D = 64
_POSITIONS = 289
_EXTENT = 8.0
_STRIDE_Y = 17.0
_NW = 32
_PAIRS_PER_W = (_B * _S) // _NW
_L = 16
_NBUF = 2
_R = 4
_GRP = _NBUF * _R
_NG = _PAIRS_PER_W // _GRP


def _sc_body(feat_hbm, tok_hbm, et_hbm, keys_w, values_w,
             keys_out, vals_out,
             feat_v, et_v, pp_v, xrow, yrow, offrow, idx, buf, gsem, wsem):
    wid = lax.axis_index("s") * 2 + lax.axis_index("c")
    b = wid // 2
    r0 = wid * _PAIRS_PER_W
    i_base = (wid % 2) * _PAIRS_PER_W

    pltpu.sync_copy(feat_hbm, feat_v)
    pltpu.sync_copy(et_hbm, et_v)
    pltpu.sync_copy(tok_hbm.at[b], pp_v)

    for c in range(_S // _L):
        tok = pp_v[pl.ds(c * _L, _L)]
        fbase = tok * _F
        xj = plsc.load_gather(feat_v, [fbase])
        yj = plsc.load_gather(feat_v, [fbase + 1])
        et = plsc.load_gather(et_v, [tok])
        xrow[pl.ds(c * _L, _L)] = xj
        yrow[pl.ds(c * _L, _L)] = yj
        offrow[pl.ds(c * _L, _L)] = et * _POSITIONS

    def compute_idx(i, p, rr, with_offset):
        ib = jnp.full((_L,), i_base + i, jnp.int32)
        xi = plsc.load_gather(xrow, [ib])
        yi = plsc.load_gather(yrow, [ib])
        for c in range(_S // _L):
            sl = pl.ds(c * _L, _L)
            dx = jnp.minimum(jnp.maximum(xrow[sl] - xi, -_EXTENT), _EXTENT)
            dy = jnp.minimum(jnp.maximum(yrow[sl] - yi, -_EXTENT), _EXTENT)
            kf = (dx + _EXTENT) + _STRIDE_Y * (dy + _EXTENT)
            kt = kf.astype(jnp.int32)
            frac = kf - kt.astype(jnp.float32)
            bump = (frac > 0.5) | ((frac == 0.5) & ((kt & 1) == 1))
            ki = kt + bump.astype(jnp.int32)
            if with_offset:
                ki = ki + offrow[sl]
            idx[p, pl.ds(rr * _S + c * _L, _L)] = ki

    def run_phase(table, out, with_offset):
        def body(g, carry):
            for p in range(_NBUF):
                i = g * _GRP + p * _R

                @pl.when(g > 0)
                def _():
                    rprev = r0 + (g - 1) * _GRP + p * _R
                    pltpu.make_async_copy(
                        buf.at[p], out.at[pl.ds(rprev * _S, _R * _S)],
                        wsem.at[p]).wait()

                for rr in range(_R):
                    compute_idx(i + rr, p, rr, with_offset)
                pltpu.async_copy(table.at[idx.at[p]], buf.at[p], gsem.at[p])
            for p in range(_NBUF):
                r = r0 + g * _GRP + p * _R
                pltpu.make_async_copy(table.at[idx.at[p]], buf.at[p],
                                      gsem.at[p]).wait()
                pltpu.async_copy(buf.at[p], out.at[pl.ds(r * _S, _R * _S)],
                                 wsem.at[p])
            return carry

        lax.fori_loop(0, _NG, body, 0)
        for p in range(_NBUF):
            rprev = r0 + (_NG - 1) * _GRP + p * _R
            pltpu.make_async_copy(buf.at[p],
                                  out.at[pl.ds(rprev * _S, _R * _S)],
                                  wsem.at[p]).wait()

    run_phase(keys_w, keys_out, False)
    run_phase(values_w, vals_out, True)


@jax.jit
def _sc_call(feat, tok, et, keys_w, values_w):
    mesh = plsc.VectorSubcoreMesh(core_axis_name="c", subcore_axis_name="s")
    f = pl.kernel(
        _sc_body,
        out_type=(
            jax.ShapeDtypeStruct((_B * _S * _S, _D), jnp.float32),
            jax.ShapeDtypeStruct((_B * _S * _S, _D), jnp.float32),
        ),
        mesh=mesh,
        compiler_params=pltpu.CompilerParams(
            needs_layout_passes=False, use_tc_tiling_on_sc=False),
        scratch_types=[
            pltpu.VMEM((_N * _F,), jnp.float32),
            pltpu.VMEM((_N,), jnp.int32),
            pltpu.VMEM((_S,), jnp.int32),
            pltpu.VMEM((_S,), jnp.float32),
            pltpu.VMEM((_S,), jnp.float32),
            pltpu.VMEM((_S,), jnp.int32),
            pltpu.VMEM((_NBUF, _R * _S), jnp.int32),
            pltpu.VMEM((_NBUF, _R * _S, _D), jnp.float32),
            pltpu.SemaphoreType.DMA((_NBUF,)),
            pltpu.SemaphoreType.DMA((_NBUF,)),
        ],
    )
    return f(feat, tok, et, keys_w, values_w)


def kernel(features, index_map, packpad_index, entity_type, keys_w, values_w):
    tok = jnp.take(index_map, packpad_index, axis=0).astype(jnp.int32)
    keys_f, vals_f = _sc_call(features.reshape(_N * _F), tok,
                              entity_type.astype(jnp.int32).reshape(_N),
                              keys_w, values_w)
    return (keys_f.reshape(_B, _S, _S, _D), vals_f.reshape(_B, _S, _S, _D))


# speedup vs baseline: 3.2207x; 3.2207x over previous
import functools

import jax
import jax.numpy as jnp
from jax import lax
from jax.experimental import pallas as pl
from jax.experimental.pallas import tpu as pltpu
from jax.experimental.pallas import tpu_sc as plsc

_B, _S, _N, _F = 16, 128, 2048, 8
_D = 64
_POSITIONS = 289
_EXTENT = 8.0
_STRIDE_Y = 17.0
_NW = 32
_PAIRS_PER_W = (_B * _S) // _NW
_L = 16
_NBUF = 4


def _sc_body(feat_hbm, tok_hbm, et_hbm, keys_w, values_w,
             keys_out, vals_out,
             feat_v, et_v, pp_v, xrow, yrow, offrow, krow, vrow,
             kbuf, vbuf, keys_s, vals_s,
             gksem, gvsem, wksem, wvsem):
    wid = lax.axis_index("s") * 2 + lax.axis_index("c")
    b = wid // 2
    r0 = wid * _PAIRS_PER_W
    i_base = (wid % 2) * _PAIRS_PER_W

    @pl.when(lax.axis_index("s") == 0)
    def _():
        pltpu.sync_copy(keys_w, keys_s)
        pltpu.sync_copy(values_w, vals_s)

    pltpu.sync_copy(feat_hbm, feat_v)
    pltpu.sync_copy(et_hbm, et_v)
    pltpu.sync_copy(tok_hbm.at[b], pp_v)

    for c in range(_S // _L):
        tok = pp_v[pl.ds(c * _L, _L)]
        fbase = tok * _F
        xj = plsc.load_gather(feat_v, [fbase])
        yj = plsc.load_gather(feat_v, [fbase + 1])
        et = plsc.load_gather(et_v, [tok])
        xrow[pl.ds(c * _L, _L)] = xj
        yrow[pl.ds(c * _L, _L)] = yj
        offrow[pl.ds(c * _L, _L)] = et * _POSITIONS

    plsc.subcore_barrier()

    def compute_idx(i, p):
        ib = jnp.full((_L,), i_base + i, jnp.int32)
        xi = plsc.load_gather(xrow, [ib])
        yi = plsc.load_gather(yrow, [ib])
        for c in range(_S // _L):
            sl = pl.ds(c * _L, _L)
            dx = jnp.minimum(jnp.maximum(xrow[sl] - xi, -_EXTENT), _EXTENT)
            dy = jnp.minimum(jnp.maximum(yrow[sl] - yi, -_EXTENT), _EXTENT)
            kf = (dx + _EXTENT) + _STRIDE_Y * (dy + _EXTENT)
            kt = kf.astype(jnp.int32)
            frac = kf - kt.astype(jnp.float32)
            bump = (frac > 0.5) | ((frac == 0.5) & ((kt & 1) == 1))
            ki = kt + bump.astype(jnp.int32)
            krow[p, sl] = ki
            vrow[p, sl] = ki + offrow[sl]

    def fire_gather(p):
        pltpu.async_copy(keys_s.at[krow.at[p]], kbuf.at[p], gksem.at[p])
        pltpu.async_copy(vals_s.at[vrow.at[p]], vbuf.at[p], gvsem.at[p])

    def drain_gather(p):
        pltpu.make_async_copy(keys_s.at[krow.at[p]], kbuf.at[p],
                              gksem.at[p]).wait()
        pltpu.make_async_copy(vals_s.at[vrow.at[p]], vbuf.at[p],
                              gvsem.at[p]).wait()

    def fire_write(r, p):
        pltpu.async_copy(kbuf.at[p], keys_out.at[pl.ds(r * _S, _S)],
                         wksem.at[p])
        pltpu.async_copy(vbuf.at[p], vals_out.at[pl.ds(r * _S, _S)],
                         wvsem.at[p])

    def drain_write(r, p):
        pltpu.make_async_copy(kbuf.at[p], keys_out.at[pl.ds(r * _S, _S)],
                              wksem.at[p]).wait()
        pltpu.make_async_copy(vbuf.at[p], vals_out.at[pl.ds(r * _S, _S)],
                              wvsem.at[p]).wait()

    n_grp = _PAIRS_PER_W // _NBUF

    def body(g, carry):
        for p in range(_NBUF):
            i = g * _NBUF + p

            @pl.when(g > 0)
            def _():
                drain_write(r0 + (g - 1) * _NBUF + p, p)

            compute_idx(i, p)
            fire_gather(p)
        for p in range(_NBUF):
            drain_gather(p)
            fire_write(r0 + g * _NBUF + p, p)
        return carry

    lax.fori_loop(0, n_grp, body, 0)
    for p in range(_NBUF):
        drain_write(r0 + (n_grp - 1) * _NBUF + p, p)


@jax.jit
def _sc_call(feat, tok, et, keys_w, values_w):
    mesh = plsc.VectorSubcoreMesh(core_axis_name="c", subcore_axis_name="s")
    f = pl.kernel(
        _sc_body,
        out_type=(
            jax.ShapeDtypeStruct((_B * _S * _S, _D), jnp.float32),
            jax.ShapeDtypeStruct((_B * _S * _S, _D), jnp.float32),
        ),
        mesh=mesh,
        compiler_params=pltpu.CompilerParams(
            needs_layout_passes=False, use_tc_tiling_on_sc=False),
        scratch_types=[
            pltpu.VMEM((_N * _F,), jnp.float32),
            pltpu.VMEM((_N,), jnp.int32),
            pltpu.VMEM((_S,), jnp.int32),
            pltpu.VMEM((_S,), jnp.float32),
            pltpu.VMEM((_S,), jnp.float32),
            pltpu.VMEM((_S,), jnp.int32),
            pltpu.VMEM((_NBUF, _S), jnp.int32),
            pltpu.VMEM((_NBUF, _S), jnp.int32),
            pltpu.VMEM((_NBUF, _S, _D), jnp.float32),
            pltpu.VMEM((_NBUF, _S, _D), jnp.float32),
            pltpu.VMEM_SHARED((_POSITIONS, _D), jnp.float32),
            pltpu.VMEM_SHARED((_POSITIONS * 4, _D), jnp.float32),
            pltpu.SemaphoreType.DMA((_NBUF,)),
            pltpu.SemaphoreType.DMA((_NBUF,)),
            pltpu.SemaphoreType.DMA((_NBUF,)),
            pltpu.SemaphoreType.DMA((_NBUF,)),
        ],
    )
    return f(feat, tok, et, keys_w, values_w)


def kernel(features, index_map, packpad_index, entity_type, keys_w, values_w):
    tok = jnp.take(index_map, packpad_index, axis=0).astype(jnp.int32)
    keys_f, vals_f = _sc_call(features.reshape(_N * _F), tok,
                              entity_type.astype(jnp.int32).reshape(_N),
                              keys_w, values_w)
    return (keys_f.reshape(_B, _S, _S, _D), vals_f.reshape(_B, _S, _S, _D))


# speedup vs baseline: 3.2237x; 1.0009x over previous
import functools

import jax
import jax.numpy as jnp
from jax import lax
from jax.experimental import pallas as pl
from jax.experimental.pallas import tpu as pltpu
from jax.experimental.pallas import tpu_sc as plsc

_B, _S, _N, _F = 16, 128, 2048, 8
_D = 64
_POSITIONS = 289
_EXTENT = 8.0
_STRIDE_Y = 17.0
_NW = 32
_PAIRS_PER_W = (_B * _S) // _NW
_L = 16
_NBUF = 4


def _sc_body(feat_hbm, tok_hbm, et_hbm, keys_w, values_w,
             keys_out, vals_out,
             feat_v, et_v, pp_v, xrow, yrow, offrow, krow, vrow,
             kbuf, vbuf, keys_s, vals_s,
             gksem, gvsem, wksem, wvsem):
    wid = lax.axis_index("s") * 2 + lax.axis_index("c")
    b = wid // 2
    r0 = wid * _PAIRS_PER_W
    i_base = (wid % 2) * _PAIRS_PER_W

    @pl.when(lax.axis_index("s") == 0)
    def _():
        pltpu.sync_copy(keys_w, keys_s)
        pltpu.sync_copy(values_w, vals_s)

    pltpu.sync_copy(feat_hbm, feat_v)
    pltpu.sync_copy(et_hbm, et_v)
    pltpu.sync_copy(tok_hbm.at[b], pp_v)

    for c in range(_S // _L):
        tok = pp_v[pl.ds(c * _L, _L)]
        fbase = tok * _F
        xj = plsc.load_gather(feat_v, [fbase])
        yj = plsc.load_gather(feat_v, [fbase + 1])
        et = plsc.load_gather(et_v, [tok])
        xrow[pl.ds(c * _L, _L)] = xj
        yrow[pl.ds(c * _L, _L)] = yj
        offrow[pl.ds(c * _L, _L)] = et * _POSITIONS

    plsc.subcore_barrier()

    def compute_idx(i, p):
        ib = jnp.full((_L,), i_base + i, jnp.int32)
        xi = plsc.load_gather(xrow, [ib])
        yi = plsc.load_gather(yrow, [ib])
        for c in range(_S // _L):
            sl = pl.ds(c * _L, _L)
            dx = jnp.minimum(jnp.maximum(xrow[sl] - xi, -_EXTENT), _EXTENT)
            dy = jnp.minimum(jnp.maximum(yrow[sl] - yi, -_EXTENT), _EXTENT)
            kf = (dx + _EXTENT) + _STRIDE_Y * (dy + _EXTENT)
            kt = kf.astype(jnp.int32)
            frac = kf - kt.astype(jnp.float32)
            bump = (frac > 0.5) | ((frac == 0.5) & ((kt & 1) == 1))
            ki = kt + bump.astype(jnp.int32)
            krow[p, sl] = ki
            vrow[p, sl] = ki + offrow[sl]

    def fire_gather(p):
        pltpu.async_copy(keys_s.at[krow.at[p]], kbuf.at[p], gksem.at[p])
        pltpu.async_copy(vals_s.at[vrow.at[p]], vbuf.at[p], gvsem.at[p])

    def drain_gather(p):
        pltpu.make_async_copy(keys_s.at[krow.at[p]], kbuf.at[p],
                              gksem.at[p]).wait()
        pltpu.make_async_copy(vals_s.at[vrow.at[p]], vbuf.at[p],
                              gvsem.at[p]).wait()

    def fire_write(i, p):
        pltpu.async_copy(kbuf.at[p], keys_out.at[b, i_base + i], wksem.at[p])
        pltpu.async_copy(vbuf.at[p], vals_out.at[b, i_base + i], wvsem.at[p])

    def drain_write(i, p):
        pltpu.make_async_copy(kbuf.at[p], keys_out.at[b, i_base + i],
                              wksem.at[p]).wait()
        pltpu.make_async_copy(vbuf.at[p], vals_out.at[b, i_base + i],
                              wvsem.at[p]).wait()

    n_grp = _PAIRS_PER_W // _NBUF

    def body(g, carry):
        for p in range(_NBUF):
            i = g * _NBUF + p

            @pl.when(g > 0)
            def _():
                drain_write((g - 1) * _NBUF + p, p)

            compute_idx(i, p)
            fire_gather(p)
        for p in range(_NBUF):
            drain_gather(p)
            fire_write(g * _NBUF + p, p)
        return carry

    lax.fori_loop(0, n_grp, body, 0)
    for p in range(_NBUF):
        drain_write((n_grp - 1) * _NBUF + p, p)


@jax.jit
def _sc_call(feat, tok, et, keys_w, values_w):
    mesh = plsc.VectorSubcoreMesh(core_axis_name="c", subcore_axis_name="s")
    f = pl.kernel(
        _sc_body,
        out_type=(
            jax.ShapeDtypeStruct((_B, _S, _S, _D), jnp.float32),
            jax.ShapeDtypeStruct((_B, _S, _S, _D), jnp.float32),
        ),
        mesh=mesh,
        compiler_params=pltpu.CompilerParams(
            needs_layout_passes=False, use_tc_tiling_on_sc=False),
        scratch_types=[
            pltpu.VMEM((_N * _F,), jnp.float32),
            pltpu.VMEM((_N,), jnp.int32),
            pltpu.VMEM((_S,), jnp.int32),
            pltpu.VMEM((_S,), jnp.float32),
            pltpu.VMEM((_S,), jnp.float32),
            pltpu.VMEM((_S,), jnp.int32),
            pltpu.VMEM((_NBUF, _S), jnp.int32),
            pltpu.VMEM((_NBUF, _S), jnp.int32),
            pltpu.VMEM((_NBUF, _S, _D), jnp.float32),
            pltpu.VMEM((_NBUF, _S, _D), jnp.float32),
            pltpu.VMEM_SHARED((_POSITIONS, _D), jnp.float32),
            pltpu.VMEM_SHARED((_POSITIONS * 4, _D), jnp.float32),
            pltpu.SemaphoreType.DMA((_NBUF,)),
            pltpu.SemaphoreType.DMA((_NBUF,)),
            pltpu.SemaphoreType.DMA((_NBUF,)),
            pltpu.SemaphoreType.DMA((_NBUF,)),
        ],
    )
    return f(feat, tok, et, keys_w, values_w)


def kernel(features, index_map, packpad_index, entity_type, keys_w, values_w):
    tok = jnp.take(index_map, packpad_index, axis=0).astype(jnp.int32)
    return _sc_call(features.reshape(_N * _F), tok,
                    entity_type.astype(jnp.int32).reshape(_N),
                    keys_w, values_w)


# speedup vs baseline: 3.2692x; 1.0141x over previous
import functools

import jax
import jax.numpy as jnp
from jax import lax
from jax.experimental import pallas as pl
from jax.experimental.pallas import tpu as pltpu
from jax.experimental.pallas import tpu_sc as plsc

_B, _S, _N, _F = 16, 128, 2048, 8
_D = 64
_POSITIONS = 289
_EXTENT = 8.0
_STRIDE_Y = 17.0
_NW = 32
_PAIRS_PER_W = (_B * _S) // _NW
_L = 16
_NBUF = 4


def _sc_body(feat_hbm, imap_hbm, tok_hbm, et_hbm, keys_w, values_w,
             keys_out, vals_out,
             feat_v, imap_v, et_v, pp_v, xrow, yrow, offrow, krow, vrow,
             kbuf, vbuf, keys_s, vals_s,
             gksem, gvsem, wksem, wvsem):
    wid = lax.axis_index("s") * 2 + lax.axis_index("c")
    b = wid // 2
    r0 = wid * _PAIRS_PER_W
    i_base = (wid % 2) * _PAIRS_PER_W

    @pl.when(lax.axis_index("s") == 0)
    def _():
        pltpu.sync_copy(keys_w, keys_s)
        pltpu.sync_copy(values_w, vals_s)

    pltpu.sync_copy(feat_hbm, feat_v)
    pltpu.sync_copy(imap_hbm, imap_v)
    pltpu.sync_copy(et_hbm, et_v)
    pltpu.sync_copy(tok_hbm.at[b], pp_v)

    for c in range(_S // _L):
        tok = plsc.load_gather(imap_v, [pp_v[pl.ds(c * _L, _L)]])
        fbase = tok * _F
        xj = plsc.load_gather(feat_v, [fbase])
        yj = plsc.load_gather(feat_v, [fbase + 1])
        et = plsc.load_gather(et_v, [tok])
        xrow[pl.ds(c * _L, _L)] = xj
        yrow[pl.ds(c * _L, _L)] = yj
        offrow[pl.ds(c * _L, _L)] = et * _POSITIONS

    plsc.subcore_barrier()

    def compute_idx(i, p):
        ib = jnp.full((_L,), i_base + i, jnp.int32)
        xi = plsc.load_gather(xrow, [ib])
        yi = plsc.load_gather(yrow, [ib])
        for c in range(_S // _L):
            sl = pl.ds(c * _L, _L)
            dx = jnp.minimum(jnp.maximum(xrow[sl] - xi, -_EXTENT), _EXTENT)
            dy = jnp.minimum(jnp.maximum(yrow[sl] - yi, -_EXTENT), _EXTENT)
            kf = (dx + _EXTENT) + _STRIDE_Y * (dy + _EXTENT)
            kt = kf.astype(jnp.int32)
            frac = kf - kt.astype(jnp.float32)
            bump = (frac > 0.5) | ((frac == 0.5) & ((kt & 1) == 1))
            ki = kt + bump.astype(jnp.int32)
            krow[p, sl] = ki
            vrow[p, sl] = ki + offrow[sl]

    def fire_gather(p):
        pltpu.async_copy(keys_s.at[krow.at[p]], kbuf.at[p], gksem.at[p])
        pltpu.async_copy(vals_s.at[vrow.at[p]], vbuf.at[p], gvsem.at[p])

    def drain_gather(p):
        pltpu.make_async_copy(keys_s.at[krow.at[p]], kbuf.at[p],
                              gksem.at[p]).wait()
        pltpu.make_async_copy(vals_s.at[vrow.at[p]], vbuf.at[p],
                              gvsem.at[p]).wait()

    def fire_write(i, p):
        pltpu.async_copy(kbuf.at[p], keys_out.at[b, i_base + i], wksem.at[p])
        pltpu.async_copy(vbuf.at[p], vals_out.at[b, i_base + i], wvsem.at[p])

    def drain_write(i, p):
        pltpu.make_async_copy(kbuf.at[p], keys_out.at[b, i_base + i],
                              wksem.at[p]).wait()
        pltpu.make_async_copy(vbuf.at[p], vals_out.at[b, i_base + i],
                              wvsem.at[p]).wait()

    n_grp = _PAIRS_PER_W // _NBUF

    def body(g, carry):
        for p in range(_NBUF):
            i = g * _NBUF + p

            @pl.when(g > 0)
            def _():
                drain_write((g - 1) * _NBUF + p, p)

            compute_idx(i, p)
            fire_gather(p)
        for p in range(_NBUF):
            drain_gather(p)
            fire_write(g * _NBUF + p, p)
        return carry

    lax.fori_loop(0, n_grp, body, 0)
    for p in range(_NBUF):
        drain_write((n_grp - 1) * _NBUF + p, p)


@jax.jit
def _sc_call(feat, imap, tok, et, keys_w, values_w):
    mesh = plsc.VectorSubcoreMesh(core_axis_name="c", subcore_axis_name="s")
    f = pl.kernel(
        _sc_body,
        out_type=(
            jax.ShapeDtypeStruct((_B, _S, _S, _D), jnp.float32),
            jax.ShapeDtypeStruct((_B, _S, _S, _D), jnp.float32),
        ),
        mesh=mesh,
        compiler_params=pltpu.CompilerParams(
            needs_layout_passes=False, use_tc_tiling_on_sc=False),
        scratch_types=[
            pltpu.VMEM((_N * _F,), jnp.float32),
            pltpu.VMEM((_N,), jnp.int32),
            pltpu.VMEM((_N,), jnp.int32),
            pltpu.VMEM((_S,), jnp.int32),
            pltpu.VMEM((_S,), jnp.float32),
            pltpu.VMEM((_S,), jnp.float32),
            pltpu.VMEM((_S,), jnp.int32),
            pltpu.VMEM((_NBUF, _S), jnp.int32),
            pltpu.VMEM((_NBUF, _S), jnp.int32),
            pltpu.VMEM((_NBUF, _S, _D), jnp.float32),
            pltpu.VMEM((_NBUF, _S, _D), jnp.float32),
            pltpu.VMEM_SHARED((_POSITIONS, _D), jnp.float32),
            pltpu.VMEM_SHARED((_POSITIONS * 4, _D), jnp.float32),
            pltpu.SemaphoreType.DMA((_NBUF,)),
            pltpu.SemaphoreType.DMA((_NBUF,)),
            pltpu.SemaphoreType.DMA((_NBUF,)),
            pltpu.SemaphoreType.DMA((_NBUF,)),
        ],
    )
    return f(feat, imap, tok, et, keys_w, values_w)


def kernel(features, index_map, packpad_index, entity_type, keys_w, values_w):
    return _sc_call(features.reshape(_N * _F), index_map.astype(jnp.int32),
                    packpad_index.astype(jnp.int32),
                    entity_type.astype(jnp.int32).reshape(_N),
                    keys_w, values_w)


# speedup vs baseline: 5.0404x; 1.5418x over previous
import functools

import jax
import jax.numpy as jnp
from jax import lax
from jax.experimental import pallas as pl
from jax.experimental.pallas import tpu as pltpu
from jax.experimental.pallas import tpu_sc as plsc

_B, _S, _N, _F = 16, 128, 2048, 8
_D = 64
_POSITIONS = 289
_EXTENT = 8.0
_STRIDE_Y = 17.0
_NW = 32
_PAIRS_PER_W = (_B * _S) // _NW
_L = 16
_NBUF = 2
_DP = 128


def _sc_body(feat_hbm, imap_hbm, tok_hbm, et_hbm, keys_w, values_w,
             keys_out, vals_out,
             feat_v, imap_v, et_v, pp_v, xrow, yrow, offrow, krow, vrow,
             kbuf, vbuf, keys_s, vals_s,
             gksem, gvsem, wksem, wvsem):
    wid = lax.axis_index("s") * 2 + lax.axis_index("c")
    b = wid // 2
    r0 = wid * _PAIRS_PER_W
    i_base = (wid % 2) * _PAIRS_PER_W

    @pl.when(lax.axis_index("s") == 0)
    def _():
        pltpu.sync_copy(keys_w, keys_s)
        pltpu.sync_copy(values_w, vals_s)

    pltpu.sync_copy(feat_hbm, feat_v)
    pltpu.sync_copy(imap_hbm, imap_v)
    pltpu.sync_copy(et_hbm, et_v)
    pltpu.sync_copy(tok_hbm.at[b], pp_v)

    for c in range(_S // _L):
        tok = plsc.load_gather(imap_v, [pp_v[pl.ds(c * _L, _L)]])
        fbase = tok * _F
        xj = plsc.load_gather(feat_v, [fbase])
        yj = plsc.load_gather(feat_v, [fbase + 1])
        et = plsc.load_gather(et_v, [tok])
        xrow[pl.ds(c * _L, _L)] = xj
        yrow[pl.ds(c * _L, _L)] = yj
        offrow[pl.ds(c * _L, _L)] = et * _POSITIONS

    plsc.subcore_barrier()

    def compute_idx(i, p):
        ib = jnp.full((_L,), i_base + i, jnp.int32)
        xi = plsc.load_gather(xrow, [ib])
        yi = plsc.load_gather(yrow, [ib])
        for c in range(_S // _L):
            sl = pl.ds(c * _L, _L)
            dx = jnp.minimum(jnp.maximum(xrow[sl] - xi, -_EXTENT), _EXTENT)
            dy = jnp.minimum(jnp.maximum(yrow[sl] - yi, -_EXTENT), _EXTENT)
            kf = (dx + _EXTENT) + _STRIDE_Y * (dy + _EXTENT)
            kt = kf.astype(jnp.int32)
            frac = kf - kt.astype(jnp.float32)
            bump = (frac > 0.5) | ((frac == 0.5) & ((kt & 1) == 1))
            ki = kt + bump.astype(jnp.int32)
            krow[p, sl] = ki
            vrow[p, sl] = ki + offrow[sl]

    def fire_gather(p):
        pltpu.async_copy(keys_s.at[krow.at[p]], kbuf.at[p], gksem.at[p])
        pltpu.async_copy(vals_s.at[vrow.at[p]], vbuf.at[p], gvsem.at[p])

    def drain_gather(p):
        pltpu.make_async_copy(keys_s.at[krow.at[p]], kbuf.at[p],
                              gksem.at[p]).wait()
        pltpu.make_async_copy(vals_s.at[vrow.at[p]], vbuf.at[p],
                              gvsem.at[p]).wait()

    def fire_write(i, p):
        g0 = (r0 + i) * _S
        pltpu.async_copy(kbuf.at[p], keys_out.at[pl.ds(g0, _S)], wksem.at[p])
        pltpu.async_copy(vbuf.at[p], vals_out.at[pl.ds(g0, _S)], wvsem.at[p])

    def drain_write(i, p):
        g0 = (r0 + i) * _S
        pltpu.make_async_copy(kbuf.at[p], keys_out.at[pl.ds(g0, _S)],
                              wksem.at[p]).wait()
        pltpu.make_async_copy(vbuf.at[p], vals_out.at[pl.ds(g0, _S)],
                              wvsem.at[p]).wait()

    n_grp = _PAIRS_PER_W // _NBUF

    def body(g, carry):
        for p in range(_NBUF):
            i = g * _NBUF + p

            @pl.when(g > 0)
            def _():
                drain_write((g - 1) * _NBUF + p, p)

            compute_idx(i, p)
            fire_gather(p)
        for p in range(_NBUF):
            drain_gather(p)
            fire_write(g * _NBUF + p, p)
        return carry

    lax.fori_loop(0, n_grp, body, 0)
    for p in range(_NBUF):
        drain_write((n_grp - 1) * _NBUF + p, p)


@jax.jit
def _sc_call(feat, imap, tok, et, keys_w, values_w):
    mesh = plsc.VectorSubcoreMesh(core_axis_name="c", subcore_axis_name="s")
    f = pl.kernel(
        _sc_body,
        out_type=(
            jax.ShapeDtypeStruct((_B * _S * _S, _DP), jnp.float32),
            jax.ShapeDtypeStruct((_B * _S * _S, _DP), jnp.float32),
        ),
        mesh=mesh,
        compiler_params=pltpu.CompilerParams(
            needs_layout_passes=False, use_tc_tiling_on_sc=False),
        scratch_types=[
            pltpu.VMEM((_N * _F,), jnp.float32),
            pltpu.VMEM((_N,), jnp.int32),
            pltpu.VMEM((_N,), jnp.int32),
            pltpu.VMEM((_S,), jnp.int32),
            pltpu.VMEM((_S,), jnp.float32),
            pltpu.VMEM((_S,), jnp.float32),
            pltpu.VMEM((_S,), jnp.int32),
            pltpu.VMEM((_NBUF, _S), jnp.int32),
            pltpu.VMEM((_NBUF, _S), jnp.int32),
            pltpu.VMEM((_NBUF, _S, _DP), jnp.float32),
            pltpu.VMEM((_NBUF, _S, _DP), jnp.float32),
            pltpu.VMEM_SHARED((_POSITIONS, _DP), jnp.float32),
            pltpu.VMEM_SHARED((_POSITIONS * 4, _DP), jnp.float32),
            pltpu.SemaphoreType.DMA((_NBUF,)),
            pltpu.SemaphoreType.DMA((_NBUF,)),
            pltpu.SemaphoreType.DMA((_NBUF,)),
            pltpu.SemaphoreType.DMA((_NBUF,)),
        ],
    )
    return f(feat, imap, tok, et, keys_w, values_w)


def kernel(features, index_map, packpad_index, entity_type, keys_w, values_w):
    keys_p = jnp.pad(keys_w, ((0, 0), (0, _DP - _D)))
    values_p = jnp.pad(values_w, ((0, 0), (0, _DP - _D)))
    kf, vf = _sc_call(features.reshape(_N * _F), index_map.astype(jnp.int32),
                      packpad_index.astype(jnp.int32),
                      entity_type.astype(jnp.int32).reshape(_N),
                      keys_p, values_p)
    return (kf[:, :_D].reshape(_B, _S, _S, _D),
            vf[:, :_D].reshape(_B, _S, _S, _D))
